# Initial kernel scaffold; baseline (speedup 1.0000x reference)
import jax
import jax.numpy as jnp
from jax.experimental import pallas as pl


def kernel(nodes, depth, edge_type, edge_name, node_type_table, node_attr_table, depth_table, edge_df_table, edge_ast_table):
    raise NotImplementedError("write your pallas kernel here")


# speedup vs baseline: 1.1047x; 1.1047x over previous
import functools

import jax
import jax.numpy as jnp
from jax import lax
from jax.experimental import pallas as pl
from jax.experimental.pallas import tpu as pltpu
from jax.experimental.pallas import tpu_sc as plsc

EMB = 128
N_NODES = 10000
N_EDGES = 320000
MAX_DEPTH = 20

_NC, _NS, _L = 2, 16, 16
_NW = _NC * _NS

_EP = 327680
_E_PER_W = _EP // _NW
_EK = 512
_EG = _EK // 128
_ENCH = _E_PER_W // _EK

_NP = 12288
_N_PER_W = _NP // _NW
_NG = _N_PER_W // 128


def _mesh():
    return plsc.VectorSubcoreMesh(
        core_axis_name="c", subcore_axis_name="s",
        num_cores=_NC, num_subcores=_NS)


def _comb_body(df_ref, ast_ref, out_ref):
    row = lax.broadcasted_iota(jnp.int32, (32, EMB), 0)
    acc = jnp.zeros((32, EMB), jnp.float32)
    for k in range(8):
        acc = acc + jnp.where(row // 4 == k, df_ref[k:k + 1, :], 0.0)
    for k in range(4):
        acc = acc + jnp.where(row % 4 == k, ast_ref[k:k + 1, :], 0.0)
    out_ref[...] = acc


def _comb_table(edge_df_table, edge_ast_table):
    return pl.pallas_call(
        _comb_body,
        out_shape=jax.ShapeDtypeStruct((32, EMB), jnp.float32),
    )(edge_df_table, edge_ast_table)


def _edge_body(et_hbm, en_hbm, comb_hbm, out_hbm, t_v, n_v, c_v, rows_v, sem):
    wid = lax.axis_index("s") * _NC + lax.axis_index("c")

    def chunk(j, carry):
        base = wid * _E_PER_W + j * _EK
        pltpu.sync_copy(et_hbm.at[pl.ds(base, _EK)], t_v)
        pltpu.sync_copy(en_hbm.at[pl.ds(base, _EK)], n_v)
        for i in range(_EK // _L):
            sl = pl.ds(i * _L, _L)
            c_v[sl] = t_v[sl] * 4 + n_v[sl]
        descs = [
            pltpu.async_copy(comb_hbm.at[c_v.at[pl.ds(g * 128, 128)]],
                             rows_v.at[pl.ds(g * 128, 128)], sem)
            for g in range(_EG)
        ]
        for d in descs:
            d.wait()
        pltpu.sync_copy(rows_v, out_hbm.at[pl.ds(base, _EK)])
        return carry

    lax.fori_loop(0, _ENCH, chunk, 0)


def _edge_sc(et2, en2, comb):
    f = functools.partial(
        pl.kernel,
        out_type=jax.ShapeDtypeStruct((_EP, EMB), jnp.float32),
        mesh=_mesh(),
        scratch_types=[
            pltpu.VMEM((_EK,), jnp.int32),
            pltpu.VMEM((_EK,), jnp.int32),
            pltpu.VMEM((_EK,), jnp.int32),
            pltpu.VMEM((_EK, EMB), jnp.float32),
            pltpu.SemaphoreType.DMA,
        ],
    )(_edge_body)
    return f(et2, en2, comb)


def _acc_add(acc, tmp):
    def row(r, carry):
        for i in range(8):
            sl = pl.ds(i * _L, _L)
            acc[r, sl] = acc[r, sl] + tmp[r, sl]
        return carry
    lax.fori_loop(0, _N_PER_W, row, 0)


def _node_body(nt_hbm, na_hbm, dp_hbm, ttab_hbm, atab_hbm, dtab_hbm, out_hbm,
               ti, ai, di, acc, tmp, sem):
    wid = lax.axis_index("s") * _NC + lax.axis_index("c")
    base = wid * _N_PER_W
    pltpu.sync_copy(nt_hbm.at[pl.ds(base, _N_PER_W)], ti)
    pltpu.sync_copy(na_hbm.at[pl.ds(base, _N_PER_W)], ai)
    pltpu.sync_copy(dp_hbm.at[pl.ds(base, _N_PER_W)], di)
    for i in range(_N_PER_W // _L):
        sl = pl.ds(i * _L, _L)
        di[sl] = jnp.minimum(di[sl], MAX_DEPTH)
    descs = [
        pltpu.async_copy(atab_hbm.at[ai.at[pl.ds(g * 128, 128)]],
                         acc.at[pl.ds(g * 128, 128)], sem)
        for g in range(_NG)
    ] + [
        pltpu.async_copy(ttab_hbm.at[ti.at[pl.ds(g * 128, 128)]],
                         tmp.at[pl.ds(g * 128, 128)], sem)
        for g in range(_NG)
    ]
    for d in descs:
        d.wait()
    _acc_add(acc, tmp)
    descs = [
        pltpu.async_copy(dtab_hbm.at[di.at[pl.ds(g * 128, 128)]],
                         tmp.at[pl.ds(g * 128, 128)], sem)
        for g in range(_NG)
    ]
    for d in descs:
        d.wait()
    _acc_add(acc, tmp)
    pltpu.sync_copy(acc, out_hbm.at[pl.ds(base, _N_PER_W)])


def _node_sc(nt2, na2, dp2, ttab, atab, dtab):
    f = functools.partial(
        pl.kernel,
        out_type=jax.ShapeDtypeStruct((_NP, EMB), jnp.float32),
        mesh=_mesh(),
        scratch_types=[
            pltpu.VMEM((_N_PER_W,), jnp.int32),
            pltpu.VMEM((_N_PER_W,), jnp.int32),
            pltpu.VMEM((_N_PER_W,), jnp.int32),
            pltpu.VMEM((_N_PER_W, EMB), jnp.float32),
            pltpu.VMEM((_N_PER_W, EMB), jnp.float32),
            pltpu.SemaphoreType.DMA,
        ],
    )(_node_body)
    return f(nt2, na2, dp2, ttab, atab, dtab)


def _pad1d(x, total):
    return jnp.pad(x, (0, total - x.shape[0]))


def kernel(nodes, depth, edge_type, edge_name, node_type_table,
           node_attr_table, depth_table, edge_df_table, edge_ast_table):
    nt2 = _pad1d(nodes[:, 0].astype(jnp.int32), _NP)
    na2 = _pad1d(nodes[:, 1].astype(jnp.int32), _NP)
    dp2 = _pad1d(depth[:, 0].astype(jnp.int32), _NP)
    et2 = _pad1d(edge_type.astype(jnp.int32), _EP)
    en2 = _pad1d(edge_name.astype(jnp.int32), _EP)

    comb = _comb_table(edge_df_table, edge_ast_table)
    edge_out = _edge_sc(et2, en2, comb)
    node_out = _node_sc(nt2, na2, dp2, node_type_table,
                        node_attr_table, depth_table)
    return (node_out[:N_NODES], edge_out[:N_EDGES])


# speedup vs baseline: 1.6451x; 1.4892x over previous
import functools

import jax
import jax.numpy as jnp
from jax import lax
from jax.experimental import pallas as pl
from jax.experimental.pallas import tpu as pltpu
from jax.experimental.pallas import tpu_sc as plsc

EMB = 128
N_NODES = 10000
N_EDGES = 320000
MAX_DEPTH = 20

_NC, _NS, _L = 2, 16, 16
_NW = _NC * _NS

_E_PER_W = N_EDGES // _NW
_EK = 400
_ENCH = _E_PER_W // _EK
_SPLITS = ((0, 128), (128, 128), (256, 128), (384, 16))

_NP = 12288
_N_PER_W = _NP // _NW
_NSPLITS = ((0, 128), (128, 128), (256, 128))


def _mesh():
    return plsc.VectorSubcoreMesh(
        core_axis_name="c", subcore_axis_name="s",
        num_cores=_NC, num_subcores=_NS)


def _comb_body(df_ref, ast_ref, out_ref):
    row = lax.broadcasted_iota(jnp.int32, (32, EMB), 0)
    acc = jnp.zeros((32, EMB), jnp.float32)
    for k in range(8):
        acc = acc + jnp.where(row // 4 == k, df_ref[k:k + 1, :], 0.0)
    for k in range(4):
        acc = acc + jnp.where(row % 4 == k, ast_ref[k:k + 1, :], 0.0)
    out_ref[...] = acc


def _comb_table(edge_df_table, edge_ast_table):
    return pl.pallas_call(
        _comb_body,
        out_shape=jax.ShapeDtypeStruct((32, EMB), jnp.float32),
    )(edge_df_table, edge_ast_table)


def _edge_body(et_hbm, en_hbm, comb_hbm, out_hbm,
               t_v, n_v, c_v, rows_v, isem, gsem, osem):
    wid = lax.axis_index("s") * _NC + lax.axis_index("c")
    base_w = wid * _E_PER_W

    def load_idx(chunk, off, sem):
        nbase = base_w + chunk * _EK
        a = pltpu.async_copy(et_hbm.at[pl.ds(nbase, _EK)],
                             t_v.at[pl.ds(off, _EK)], sem)
        b = pltpu.async_copy(en_hbm.at[pl.ds(nbase, _EK)],
                             n_v.at[pl.ds(off, _EK)], sem)
        return a, b

    def compute_cidx(off):
        for i in range(_EK // _L):
            sl = pl.ds(off + i * _L, _L)
            c_v[sl] = t_v[sl] * 4 + n_v[sl]

    def issue_gathers(off):
        return [
            pltpu.async_copy(comb_hbm.at[c_v.at[pl.ds(off + o, ln)]],
                             rows_v.at[pl.ds(off + o, ln)], gsem)
            for (o, ln) in _SPLITS
        ]

    def wait_gathers(off):
        for (o, ln) in _SPLITS:
            pltpu.make_async_copy(comb_hbm.at[c_v.at[pl.ds(off + o, ln)]],
                                  rows_v.at[pl.ds(off + o, ln)], gsem).wait()

    def wait_write(off):
        pltpu.make_async_copy(rows_v.at[pl.ds(off, _EK)],
                              out_hbm.at[pl.ds(base_w, _EK)], osem).wait()

    for d in load_idx(0, 0, isem):
        d.wait()
    compute_cidx(0)
    issue_gathers(0)

    def body(j, carry):
        p = j % 2
        off_p = p * _EK
        off_q = (1 - p) * _EK
        nxt = j + 1

        @pl.when(nxt < _ENCH)
        def _():
            load_idx(nxt, off_q, isem)

        wait_gathers(off_p)

        @pl.when(j >= 1)
        def _():
            wait_write(off_q)

        @pl.when(nxt < _ENCH)
        def _():
            pltpu.make_async_copy(et_hbm.at[pl.ds(base_w, _EK)],
                                  t_v.at[pl.ds(off_q, _EK)], isem).wait()
            pltpu.make_async_copy(en_hbm.at[pl.ds(base_w, _EK)],
                                  n_v.at[pl.ds(off_q, _EK)], isem).wait()
            compute_cidx(off_q)
            issue_gathers(off_q)

        pltpu.async_copy(rows_v.at[pl.ds(off_p, _EK)],
                         out_hbm.at[pl.ds(base_w + j * _EK, _EK)], osem)
        return carry

    lax.fori_loop(0, _ENCH, body, 0)
    pltpu.make_async_copy(rows_v.at[pl.ds(0, _EK)],
                          out_hbm.at[pl.ds(base_w, _EK)], osem).wait()


def _edge_sc(et, en, comb):
    f = functools.partial(
        pl.kernel,
        out_type=jax.ShapeDtypeStruct((N_EDGES, EMB), jnp.float32),
        mesh=_mesh(),
        scratch_types=[
            pltpu.VMEM((2 * _EK,), jnp.int32),
            pltpu.VMEM((2 * _EK,), jnp.int32),
            pltpu.VMEM((2 * _EK,), jnp.int32),
            pltpu.VMEM((2 * _EK, EMB), jnp.float32),
            pltpu.SemaphoreType.DMA,
            pltpu.SemaphoreType.DMA,
            pltpu.SemaphoreType.DMA,
        ],
    )(_edge_body)
    return f(et, en, comb)


def _acc_add(acc, tmp):
    def rows8(j, carry):
        for k in range(8):
            r = j * 8 + k
            for i in range(8):
                sl = pl.ds(i * _L, _L)
                acc[r, sl] = acc[r, sl] + tmp[r, sl]
        return carry
    lax.fori_loop(0, _N_PER_W // 8, rows8, 0)


def _node_body(nt_hbm, na_hbm, dp_hbm, ttab_hbm, atab_hbm, dtab_hbm, out_hbm,
               ti, ai, di, acc, tmp, sem):
    wid = lax.axis_index("s") * _NC + lax.axis_index("c")
    base = wid * _N_PER_W
    pltpu.sync_copy(nt_hbm.at[pl.ds(base, _N_PER_W)], ti)
    pltpu.sync_copy(na_hbm.at[pl.ds(base, _N_PER_W)], ai)
    pltpu.sync_copy(dp_hbm.at[pl.ds(base, _N_PER_W)], di)
    for i in range(_N_PER_W // _L):
        sl = pl.ds(i * _L, _L)
        di[sl] = jnp.minimum(di[sl], MAX_DEPTH)
    descs = [
        pltpu.async_copy(atab_hbm.at[ai.at[pl.ds(o, ln)]],
                         acc.at[pl.ds(o, ln)], sem)
        for (o, ln) in _NSPLITS
    ] + [
        pltpu.async_copy(ttab_hbm.at[ti.at[pl.ds(o, ln)]],
                         tmp.at[pl.ds(o, ln)], sem)
        for (o, ln) in _NSPLITS
    ]
    for d in descs:
        d.wait()
    _acc_add(acc, tmp)
    descs = [
        pltpu.async_copy(dtab_hbm.at[di.at[pl.ds(o, ln)]],
                         tmp.at[pl.ds(o, ln)], sem)
        for (o, ln) in _NSPLITS
    ]
    for d in descs:
        d.wait()
    _acc_add(acc, tmp)
    pltpu.sync_copy(acc, out_hbm.at[pl.ds(base, _N_PER_W)])


def _node_sc(nt2, na2, dp2, ttab, atab, dtab):
    f = functools.partial(
        pl.kernel,
        out_type=jax.ShapeDtypeStruct((_NP, EMB), jnp.float32),
        mesh=_mesh(),
        scratch_types=[
            pltpu.VMEM((_N_PER_W,), jnp.int32),
            pltpu.VMEM((_N_PER_W,), jnp.int32),
            pltpu.VMEM((_N_PER_W,), jnp.int32),
            pltpu.VMEM((_N_PER_W, EMB), jnp.float32),
            pltpu.VMEM((_N_PER_W, EMB), jnp.float32),
            pltpu.SemaphoreType.DMA,
        ],
    )(_node_body)
    return f(nt2, na2, dp2, ttab, atab, dtab)


def _pad1d(x, total):
    return jnp.pad(x, (0, total - x.shape[0]))


def kernel(nodes, depth, edge_type, edge_name, node_type_table,
           node_attr_table, depth_table, edge_df_table, edge_ast_table):
    nt2 = _pad1d(nodes[:, 0].astype(jnp.int32), _NP)
    na2 = _pad1d(nodes[:, 1].astype(jnp.int32), _NP)
    dp2 = _pad1d(depth[:, 0].astype(jnp.int32), _NP)
    et = edge_type.astype(jnp.int32)
    en = edge_name.astype(jnp.int32)

    comb = _comb_table(edge_df_table, edge_ast_table)
    edge_out = _edge_sc(et, en, comb)
    node_out = _node_sc(nt2, na2, dp2, node_type_table,
                        node_attr_table, depth_table)
    return (node_out[:N_NODES], edge_out)


# speedup vs baseline: 1.9013x; 1.1557x over previous
import functools

import jax
import jax.numpy as jnp
from jax import lax
from jax.experimental import pallas as pl
from jax.experimental.pallas import tpu as pltpu
from jax.experimental.pallas import tpu_sc as plsc

EMB = 128
N_NODES = 10000
N_EDGES = 320000
MAX_DEPTH = 20

_NC, _NS, _L = 2, 16, 16
_NW = _NC * _NS

_E_PER_W = N_EDGES // _NW
_EK = 400
_ENCH = _E_PER_W // _EK
_SPLITS = ((0, 128), (128, 128), (256, 128), (384, 16))

_NP = 12288
_N_PER_W = _NP // _NW
_NSPLITS = ((0, 128), (128, 128), (256, 128))

_NB = 2000
_TT_PAD = 104
_DT_PAD = 24


def _mesh():
    return plsc.VectorSubcoreMesh(
        core_axis_name="c", subcore_axis_name="s",
        num_cores=_NC, num_subcores=_NS)


def _comb_body(df_ref, ast_ref, out_ref):
    row = lax.broadcasted_iota(jnp.int32, (32, EMB), 0)
    acc = jnp.zeros((32, EMB), jnp.float32)
    for k in range(8):
        acc = acc + jnp.where(row // 4 == k, df_ref[k:k + 1, :], 0.0)
    for k in range(4):
        acc = acc + jnp.where(row % 4 == k, ast_ref[k:k + 1, :], 0.0)
    out_ref[...] = acc


def _comb_table(edge_df_table, edge_ast_table):
    return pl.pallas_call(
        _comb_body,
        out_shape=jax.ShapeDtypeStruct((32, EMB), jnp.float32),
    )(edge_df_table, edge_ast_table)


def _edge_body(et_hbm, en_hbm, comb_hbm, out_hbm,
               t_v, n_v, c_v, rows_v, isem, gsem0, gsem1, osem):
    wid = lax.axis_index("s") * _NC + lax.axis_index("c")
    base_w = wid * _E_PER_W

    def load_idx(chunk, off):
        nbase = base_w + chunk * _EK
        pltpu.async_copy(et_hbm.at[pl.ds(nbase, _EK)],
                         t_v.at[pl.ds(off, _EK)], isem)
        pltpu.async_copy(en_hbm.at[pl.ds(nbase, _EK)],
                         n_v.at[pl.ds(off, _EK)], isem)

    def wait_idx(off):
        pltpu.make_async_copy(et_hbm.at[pl.ds(base_w, _EK)],
                              t_v.at[pl.ds(off, _EK)], isem).wait()
        pltpu.make_async_copy(en_hbm.at[pl.ds(base_w, _EK)],
                              n_v.at[pl.ds(off, _EK)], isem).wait()

    def compute_cidx(off):
        for i in range(_EK // _L):
            sl = pl.ds(off + i * _L, _L)
            c_v[sl] = t_v[sl] * 4 + n_v[sl]

    def issue_gathers(off, gsem):
        for (o, ln) in _SPLITS:
            pltpu.async_copy(comb_hbm.at[c_v.at[pl.ds(off + o, ln)]],
                             rows_v.at[pl.ds(off + o, ln)], gsem)

    def wait_gathers(off, gsem):
        for (o, ln) in _SPLITS:
            pltpu.make_async_copy(comb_hbm.at[c_v.at[pl.ds(off + o, ln)]],
                                  rows_v.at[pl.ds(off + o, ln)], gsem).wait()

    def wait_write(off):
        pltpu.make_async_copy(rows_v.at[pl.ds(off, _EK)],
                              out_hbm.at[pl.ds(base_w, _EK)], osem).wait()

    load_idx(0, 0)
    wait_idx(0)
    compute_cidx(0)
    issue_gathers(0, gsem0)
    load_idx(1, _EK)

    def step(j, off_p, off_q, gsem_p, gsem_q):
        @pl.when(j >= 1)
        def _():
            wait_write(off_q)

        @pl.when(j + 1 < _ENCH)
        def _():
            wait_idx(off_q)
            compute_cidx(off_q)
            issue_gathers(off_q, gsem_q)

        @pl.when(j + 2 < _ENCH)
        def _():
            load_idx(j + 2, off_p)

        wait_gathers(off_p, gsem_p)
        pltpu.async_copy(rows_v.at[pl.ds(off_p, _EK)],
                         out_hbm.at[pl.ds(base_w + j * _EK, _EK)], osem)

    def body(j, carry):
        @pl.when(j % 2 == 0)
        def _():
            step(j, 0, _EK, gsem0, gsem1)

        @pl.when(j % 2 == 1)
        def _():
            step(j, _EK, 0, gsem1, gsem0)
        return carry

    lax.fori_loop(0, _ENCH, body, 0)
    wait_write(0)


def _edge_sc(et, en, comb):
    f = functools.partial(
        pl.kernel,
        out_type=jax.ShapeDtypeStruct((N_EDGES, EMB), jnp.float32),
        mesh=_mesh(),
        scratch_types=[
            pltpu.VMEM((2 * _EK,), jnp.int32),
            pltpu.VMEM((2 * _EK,), jnp.int32),
            pltpu.VMEM((2 * _EK,), jnp.int32),
            pltpu.VMEM((2 * _EK, EMB), jnp.float32),
            pltpu.SemaphoreType.DMA,
            pltpu.SemaphoreType.DMA,
            pltpu.SemaphoreType.DMA,
            pltpu.SemaphoreType.DMA,
        ],
    )(_edge_body)
    return f(et, en, comb)


def _nattr_body(ai_hbm, atab_hbm, out_hbm, ai, buf, sem):
    wid = lax.axis_index("s") * _NC + lax.axis_index("c")
    base = wid * _N_PER_W
    pltpu.sync_copy(ai_hbm.at[pl.ds(base, _N_PER_W)], ai)
    descs = [
        pltpu.async_copy(atab_hbm.at[ai.at[pl.ds(o, ln)]],
                         buf.at[pl.ds(o, ln)], sem)
        for (o, ln) in _NSPLITS
    ]
    for d in descs:
        d.wait()
    pltpu.sync_copy(buf, out_hbm.at[pl.ds(base, _N_PER_W)])


def _nattr_sc(na2, atab):
    f = functools.partial(
        pl.kernel,
        out_type=jax.ShapeDtypeStruct((_NP, EMB), jnp.float32),
        mesh=_mesh(),
        scratch_types=[
            pltpu.VMEM((_N_PER_W,), jnp.int32),
            pltpu.VMEM((_N_PER_W, EMB), jnp.float32),
            pltpu.SemaphoreType.DMA,
        ],
    )(_nattr_body)
    return f(na2, atab)


def _nodesum_body(attr_ref, nt_ref, dp_ref, ttab_ref, dtab_ref, out_ref):
    ntv = nt_ref[0]
    dpv = jnp.minimum(dp_ref[0], MAX_DEPTH)
    kt = lax.broadcasted_iota(jnp.int32, (_TT_PAD, _NB), 0)
    hot_t = (kt == ntv).astype(jnp.float32)
    kd = lax.broadcasted_iota(jnp.int32, (_DT_PAD, _NB), 0)
    hot_d = (kd == dpv).astype(jnp.float32)
    dn = (((0,), (0,)), ((), ()))
    te = lax.dot_general(hot_t, ttab_ref[...], dn,
                         precision=lax.Precision.HIGHEST,
                         preferred_element_type=jnp.float32)
    de = lax.dot_general(hot_d, dtab_ref[...], dn,
                         precision=lax.Precision.HIGHEST,
                         preferred_element_type=jnp.float32)
    out_ref[...] = attr_ref[...] + te + de


def _node_sum_tc(attr_rows, nt3, dp3, ttab_p, dtab_p):
    grid = (N_NODES // _NB,)
    return pl.pallas_call(
        _nodesum_body,
        grid=grid,
        in_specs=[
            pl.BlockSpec((_NB, EMB), lambda i: (i, 0)),
            pl.BlockSpec((1, 1, _NB), lambda i: (i, 0, 0)),
            pl.BlockSpec((1, 1, _NB), lambda i: (i, 0, 0)),
            pl.BlockSpec((_TT_PAD, EMB), lambda i: (0, 0)),
            pl.BlockSpec((_DT_PAD, EMB), lambda i: (0, 0)),
        ],
        out_specs=pl.BlockSpec((_NB, EMB), lambda i: (i, 0)),
        out_shape=jax.ShapeDtypeStruct((N_NODES, EMB), jnp.float32),
    )(attr_rows, nt3, dp3, ttab_p, dtab_p)


def _pad1d(x, total):
    return jnp.pad(x, (0, total - x.shape[0]))


def kernel(nodes, depth, edge_type, edge_name, node_type_table,
           node_attr_table, depth_table, edge_df_table, edge_ast_table):
    na2 = _pad1d(nodes[:, 1].astype(jnp.int32), _NP)
    nt3 = nodes[:, 0].astype(jnp.int32).reshape(N_NODES // _NB, 1, _NB)
    dp3 = depth[:, 0].astype(jnp.int32).reshape(N_NODES // _NB, 1, _NB)
    et = edge_type.astype(jnp.int32)
    en = edge_name.astype(jnp.int32)
    ttab_p = jnp.pad(node_type_table, ((0, _TT_PAD - 100), (0, 0)))
    dtab_p = jnp.pad(depth_table, ((0, _DT_PAD - (MAX_DEPTH + 1)), (0, 0)))

    attr_rows = _nattr_sc(na2, node_attr_table)
    comb = _comb_table(edge_df_table, edge_ast_table)
    edge_out = _edge_sc(et, en, comb)
    node_out = _node_sum_tc(attr_rows, nt3, dp3, ttab_p, dtab_p)
    return (node_out, edge_out)


# speedup vs baseline: 7.0834x; 3.7256x over previous
import functools

import jax
import jax.numpy as jnp
from jax import lax
from jax.experimental import pallas as pl
from jax.experimental.pallas import tpu as pltpu
from jax.experimental.pallas import tpu_sc as plsc

EMB = 128
N_NODES = 10000
N_EDGES = 320000
MAX_DEPTH = 20

_NC, _NS, _L = 2, 16, 16
_NW = _NC * _NS

_EB = 384
_ECH = 26
_EW = _EB * _ECH
_EQ = _EW // 4
_TAIL_BASE = _EW * _NW
_TAIL = N_EDGES - _TAIL_BASE
_OUT3 = N_EDGES // 128

_NP = 12288
_N_PER_W = _NP // _NW

_NB = 2000
_TT_PAD = 104
_DT_PAD = 24


def _mesh():
    return plsc.VectorSubcoreMesh(
        core_axis_name="c", subcore_axis_name="s",
        num_cores=_NC, num_subcores=_NS)


def _comb_body(df_ref, ast_ref, out_ref):
    row = lax.broadcasted_iota(jnp.int32, (32, EMB), 0)
    acc = jnp.zeros((32, EMB), jnp.float32)
    for k in range(8):
        acc = acc + jnp.where(row // 4 == k, df_ref[k:k + 1, :], 0.0)
    for k in range(4):
        acc = acc + jnp.where(row % 4 == k, ast_ref[k:k + 1, :], 0.0)
    out_ref[...] = acc


def _comb_table(edge_df_table, edge_ast_table):
    return pl.pallas_call(
        _comb_body,
        out_shape=jax.ShapeDtypeStruct((32, EMB), jnp.float32),
    )(edge_df_table, edge_ast_table)


def _edge_body(et_hbm, en_hbm, comb_hbm, out_hbm,
               t_q, n_q, c_v, t_t, n_t, c_t, rows_v, comb_sh,
               isem, gsem0, gsem1, osem):
    wid = lax.axis_index("s") * _NC + lax.axis_index("c")
    base_w = wid * _EW
    obase = wid * (_EW // 128)

    @pl.when(lax.axis_index("s") == 0)
    def _():
        pltpu.sync_copy(comb_hbm, comb_sh)

    for qr in range(4):
        qoff = qr * _EQ
        a = pltpu.async_copy(et_hbm.at[pl.ds(base_w + qoff, _EQ)], t_q, isem)
        b = pltpu.async_copy(en_hbm.at[pl.ds(base_w + qoff, _EQ)], n_q, isem)
        a.wait()
        b.wait()

        def cbody(i, carry, qoff=qoff):
            sl = pl.ds(i * _L, _L)
            c_v[pl.ds(qoff + i * _L, _L)] = t_q[sl] * 4 + n_q[sl]
            return carry
        lax.fori_loop(0, _EQ // _L, cbody, 0)

    plsc.subcore_barrier()

    @pl.when(wid == _NW - 1)
    def _():
        a = pltpu.async_copy(et_hbm.at[pl.ds(_TAIL_BASE, _TAIL)], t_t, isem)
        b = pltpu.async_copy(en_hbm.at[pl.ds(_TAIL_BASE, _TAIL)], n_t, isem)
        a.wait()
        b.wait()

        def tbody(i, carry):
            sl = pl.ds(i * _L, _L)
            c_t[sl] = t_t[sl] * 4 + n_t[sl]
            return carry
        lax.fori_loop(0, _TAIL // _L, tbody, 0)
        pltpu.async_copy(comb_sh.at[c_t.at[pl.ds(0, _EB)]],
                         rows_v.at[0], gsem0).wait()
        pltpu.sync_copy(rows_v.at[0],
                        out_hbm.at[pl.ds(_TAIL_BASE, _EB)])
        pltpu.async_copy(comb_sh.at[c_t.at[pl.ds(_EB, 128)]],
                         rows_v.at[0, pl.ds(0, 128)], gsem0).wait()
        pltpu.sync_copy(rows_v.at[0, pl.ds(0, 128)],
                        out_hbm.at[pl.ds(_TAIL_BASE + _EB, 128)])

    def issue_gather(j, slot, gsem):
        pltpu.async_copy(comb_sh.at[c_v.at[pl.ds(_EB * j, _EB)]],
                         rows_v.at[slot], gsem)

    def wait_gather(slot, gsem):
        pltpu.make_async_copy(comb_sh.at[c_v.at[pl.ds(0, _EB)]],
                              rows_v.at[slot], gsem).wait()

    def issue_write(j, slot):
        pltpu.async_copy(rows_v.at[slot],
                         out_hbm.at[pl.ds(base_w + _EB * j, _EB)], osem)

    def wait_write(slot):
        pltpu.make_async_copy(rows_v.at[slot],
                              out_hbm.at[pl.ds(0, _EB)], osem).wait()

    issue_gather(0, 0, gsem0)

    def step(j, sp, sq, gsem_p, gsem_q):
        @pl.when(j >= 1)
        def _():
            wait_write(sq)

        @pl.when(j + 1 < _ECH)
        def _():
            issue_gather(j + 1, sq, gsem_q)

        wait_gather(sp, gsem_p)
        issue_write(j, sp)

    def body(j, carry):
        @pl.when(j % 2 == 0)
        def _():
            step(j, 0, 1, gsem0, gsem1)

        @pl.when(j % 2 == 1)
        def _():
            step(j, 1, 0, gsem1, gsem0)
        return carry

    lax.fori_loop(0, _ECH, body, 0)
    wait_write(1)


def _edge_sc(et, en, comb):
    f = functools.partial(
        pl.kernel,
        out_type=jax.ShapeDtypeStruct((N_EDGES, EMB), jnp.float32),
        mesh=_mesh(),
        scratch_types=[
            pltpu.VMEM((_EQ,), jnp.int32),
            pltpu.VMEM((_EQ,), jnp.int32),
            pltpu.VMEM((_EW,), jnp.int32),
            pltpu.VMEM((_TAIL,), jnp.int32),
            pltpu.VMEM((_TAIL,), jnp.int32),
            pltpu.VMEM((_TAIL,), jnp.int32),
            pltpu.VMEM((2, _EB, EMB), jnp.float32),
            pltpu.VMEM_SHARED((32, EMB), jnp.float32),
            pltpu.SemaphoreType.DMA,
            pltpu.SemaphoreType.DMA,
            pltpu.SemaphoreType.DMA,
            pltpu.SemaphoreType.DMA,
        ],
    )(_edge_body)
    return f(et, en, comb)


def _nattr_body(na_hbm, atab_hbm, out_hbm, ai, buf, sem):
    wid = lax.axis_index("s") * _NC + lax.axis_index("c")
    base = wid * _N_PER_W
    pltpu.sync_copy(na_hbm.at[pl.ds(base, _N_PER_W)], ai)
    pltpu.async_copy(atab_hbm.at[ai], buf, sem).wait()
    pltpu.sync_copy(buf, out_hbm.at[pl.ds(base, _N_PER_W)])


def _nattr_sc(na2, atab):
    f = functools.partial(
        pl.kernel,
        out_type=jax.ShapeDtypeStruct((_NP, EMB), jnp.float32),
        mesh=_mesh(),
        scratch_types=[
            pltpu.VMEM((_N_PER_W,), jnp.int32),
            pltpu.VMEM((_N_PER_W, EMB), jnp.float32),
            pltpu.SemaphoreType.DMA,
        ],
    )(_nattr_body)
    return f(na2, atab)


def _nodesum_body(attr_ref, nt_ref, dp_ref, ttab_ref, dtab_ref, out_ref):
    ntv = nt_ref[0]
    dpv = jnp.minimum(dp_ref[0], MAX_DEPTH)
    kt = lax.broadcasted_iota(jnp.int32, (_TT_PAD, _NB), 0)
    hot_t = (kt == ntv).astype(jnp.float32)
    kd = lax.broadcasted_iota(jnp.int32, (_DT_PAD, _NB), 0)
    hot_d = (kd == dpv).astype(jnp.float32)
    dn = (((0,), (0,)), ((), ()))
    te = lax.dot_general(hot_t, ttab_ref[...], dn,
                         precision=lax.Precision.HIGHEST,
                         preferred_element_type=jnp.float32)
    de = lax.dot_general(hot_d, dtab_ref[...], dn,
                         precision=lax.Precision.HIGHEST,
                         preferred_element_type=jnp.float32)
    out_ref[...] = attr_ref[...] + te + de


def _node_sum_tc(attr_rows, nt3, dp3, ttab_p, dtab_p):
    grid = (N_NODES // _NB,)
    return pl.pallas_call(
        _nodesum_body,
        grid=grid,
        in_specs=[
            pl.BlockSpec((_NB, EMB), lambda i: (i, 0)),
            pl.BlockSpec((1, 1, _NB), lambda i: (i, 0, 0)),
            pl.BlockSpec((1, 1, _NB), lambda i: (i, 0, 0)),
            pl.BlockSpec((_TT_PAD, EMB), lambda i: (0, 0)),
            pl.BlockSpec((_DT_PAD, EMB), lambda i: (0, 0)),
        ],
        out_specs=pl.BlockSpec((_NB, EMB), lambda i: (i, 0)),
        out_shape=jax.ShapeDtypeStruct((N_NODES, EMB), jnp.float32),
    )(attr_rows, nt3, dp3, ttab_p, dtab_p)


def _pad1d(x, total):
    return jnp.pad(x, (0, total - x.shape[0]))


def kernel(nodes, depth, edge_type, edge_name, node_type_table,
           node_attr_table, depth_table, edge_df_table, edge_ast_table):
    na2 = _pad1d(nodes[:, 1].astype(jnp.int32), _NP)
    nt3 = nodes[:, 0].astype(jnp.int32).reshape(N_NODES // _NB, 1, _NB)
    dp3 = depth[:, 0].astype(jnp.int32).reshape(N_NODES // _NB, 1, _NB)
    et = edge_type.astype(jnp.int32)
    en = edge_name.astype(jnp.int32)
    ttab_p = jnp.pad(node_type_table, ((0, _TT_PAD - 100), (0, 0)))
    dtab_p = jnp.pad(depth_table, ((0, _DT_PAD - (MAX_DEPTH + 1)), (0, 0)))

    attr_rows = _nattr_sc(na2, node_attr_table)
    comb = _comb_table(edge_df_table, edge_ast_table)
    edge_out = _edge_sc(et, en, comb)
    node_out = _node_sum_tc(attr_rows, nt3, dp3, ttab_p, dtab_p)
    return (node_out, edge_out)


# speedup vs baseline: 7.1539x; 1.0099x over previous
import functools

import jax
import jax.numpy as jnp
from jax import lax
from jax.experimental import pallas as pl
from jax.experimental.pallas import tpu as pltpu
from jax.experimental.pallas import tpu_sc as plsc

EMB = 128
N_NODES = 10000
N_EDGES = 320000
MAX_DEPTH = 20

_NC, _NS, _L = 2, 16, 16
_NW = _NC * _NS

_EB = 256
_ECH = 39
_EW = _EB * _ECH
_EQ = _EW // 8
_TAIL_BASE = _EW * _NW
_TAIL = N_EDGES - _TAIL_BASE
_OUT3 = N_EDGES // 128

_NP = 12288
_N_PER_W = _NP // _NW

_NB = 2000
_TT_PAD = 104
_DT_PAD = 24


def _mesh():
    return plsc.VectorSubcoreMesh(
        core_axis_name="c", subcore_axis_name="s",
        num_cores=_NC, num_subcores=_NS)


def _comb_body(df_ref, ast_ref, out_ref):
    row = lax.broadcasted_iota(jnp.int32, (32, EMB), 0)
    acc = jnp.zeros((32, EMB), jnp.float32)
    for k in range(8):
        acc = acc + jnp.where(row // 4 == k, df_ref[k:k + 1, :], 0.0)
    for k in range(4):
        acc = acc + jnp.where(row % 4 == k, ast_ref[k:k + 1, :], 0.0)
    out_ref[...] = acc


def _comb_table(edge_df_table, edge_ast_table):
    return pl.pallas_call(
        _comb_body,
        out_shape=jax.ShapeDtypeStruct((32, EMB), jnp.float32),
    )(edge_df_table, edge_ast_table)


def _edge_body(et_hbm, en_hbm, comb_hbm, na_hbm, atab_hbm,
               out_hbm, attr_hbm,
               t_q, n_q, c_v, c_t, ai_v, rows_v, nrows_v, comb_sh,
               isem, gsem0, gsem1, osem, nsem):
    wid = lax.axis_index("s") * _NC + lax.axis_index("c")
    base_w = wid * _EW
    nbase = wid * _N_PER_W

    pltpu.async_copy(na_hbm.at[pl.ds(nbase, _N_PER_W)], ai_v, isem).wait()
    pltpu.async_copy(atab_hbm.at[ai_v], nrows_v, nsem)

    @pl.when(lax.axis_index("s") == 0)
    def _():
        pltpu.sync_copy(comb_hbm, comb_sh)

    for qr in range(8):
        qoff = qr * _EQ
        a = pltpu.async_copy(et_hbm.at[pl.ds(base_w + qoff, _EQ)], t_q, isem)
        b = pltpu.async_copy(en_hbm.at[pl.ds(base_w + qoff, _EQ)], n_q, isem)
        a.wait()
        b.wait()

        def cbody(i, carry, qoff=qoff):
            sl = pl.ds(i * _L, _L)
            c_v[pl.ds(qoff + i * _L, _L)] = t_q[sl] * 4 + n_q[sl]
            return carry
        lax.fori_loop(0, _EQ // _L, cbody, 0)

    plsc.subcore_barrier()

    @pl.when(wid == _NW - 1)
    def _():
        a = pltpu.async_copy(et_hbm.at[pl.ds(_TAIL_BASE, _TAIL)],
                             t_q.at[pl.ds(0, _TAIL)], isem)
        b = pltpu.async_copy(en_hbm.at[pl.ds(_TAIL_BASE, _TAIL)],
                             n_q.at[pl.ds(0, _TAIL)], isem)
        a.wait()
        b.wait()

        def tbody(i, carry):
            sl = pl.ds(i * _L, _L)
            c_t[sl] = t_q[sl] * 4 + n_q[sl]
            return carry
        lax.fori_loop(0, _TAIL // _L, tbody, 0)
        for h in range(2):
            pltpu.async_copy(comb_sh.at[c_t.at[pl.ds(h * _EB, _EB)]],
                             rows_v.at[0], gsem0).wait()
            pltpu.sync_copy(rows_v.at[0],
                            out_hbm.at[pl.ds(_TAIL_BASE + h * _EB, _EB)])

    def issue_gather(j, slot, gsem):
        pltpu.async_copy(comb_sh.at[c_v.at[pl.ds(_EB * j, _EB)]],
                         rows_v.at[slot], gsem)

    def wait_gather(slot, gsem):
        pltpu.make_async_copy(comb_sh.at[c_v.at[pl.ds(0, _EB)]],
                              rows_v.at[slot], gsem).wait()

    def issue_write(j, slot):
        pltpu.async_copy(rows_v.at[slot],
                         out_hbm.at[pl.ds(base_w + _EB * j, _EB)], osem)

    def wait_write(slot):
        pltpu.make_async_copy(rows_v.at[slot],
                              out_hbm.at[pl.ds(0, _EB)], osem).wait()

    issue_gather(0, 0, gsem0)

    def step(j, sp, sq, gsem_p, gsem_q):
        @pl.when(j >= 1)
        def _():
            wait_write(sq)

        @pl.when(j + 1 < _ECH)
        def _():
            issue_gather(j + 1, sq, gsem_q)

        wait_gather(sp, gsem_p)
        issue_write(j, sp)

    def body(j, carry):
        @pl.when(j % 2 == 0)
        def _():
            step(j, 0, 1, gsem0, gsem1)

        @pl.when(j % 2 == 1)
        def _():
            step(j, 1, 0, gsem1, gsem0)
        return carry

    lax.fori_loop(0, _ECH, body, 0)
    wait_write(1)

    pltpu.make_async_copy(atab_hbm.at[ai_v], nrows_v, nsem).wait()
    pltpu.sync_copy(nrows_v, attr_hbm.at[pl.ds(nbase, _N_PER_W)])


def _edge_sc(et, en, comb, na2, atab):
    f = functools.partial(
        pl.kernel,
        out_type=(jax.ShapeDtypeStruct((N_EDGES, EMB), jnp.float32),
                  jax.ShapeDtypeStruct((_NP, EMB), jnp.float32)),
        mesh=_mesh(),
        scratch_types=[
            pltpu.VMEM((_EQ,), jnp.int32),
            pltpu.VMEM((_EQ,), jnp.int32),
            pltpu.VMEM((_EW,), jnp.int32),
            pltpu.VMEM((_TAIL,), jnp.int32),
            pltpu.VMEM((_N_PER_W,), jnp.int32),
            pltpu.VMEM((2, _EB, EMB), jnp.float32),
            pltpu.VMEM((_N_PER_W, EMB), jnp.float32),
            pltpu.VMEM_SHARED((32, EMB), jnp.float32),
            pltpu.SemaphoreType.DMA,
            pltpu.SemaphoreType.DMA,
            pltpu.SemaphoreType.DMA,
            pltpu.SemaphoreType.DMA,
            pltpu.SemaphoreType.DMA,
        ],
    )(_edge_body)
    return f(et, en, comb, na2, atab)


def _nodesum_body(attr_ref, nt_ref, dp_ref, ttab_ref, dtab_ref, out_ref):
    ntv = nt_ref[0]
    dpv = jnp.minimum(dp_ref[0], MAX_DEPTH)
    kt = lax.broadcasted_iota(jnp.int32, (_TT_PAD, _NB), 0)
    hot_t = (kt == ntv).astype(jnp.float32)
    kd = lax.broadcasted_iota(jnp.int32, (_DT_PAD, _NB), 0)
    hot_d = (kd == dpv).astype(jnp.float32)
    dn = (((0,), (0,)), ((), ()))
    te = lax.dot_general(hot_t, ttab_ref[...], dn,
                         precision=lax.Precision.HIGHEST,
                         preferred_element_type=jnp.float32)
    de = lax.dot_general(hot_d, dtab_ref[...], dn,
                         precision=lax.Precision.HIGHEST,
                         preferred_element_type=jnp.float32)
    out_ref[...] = attr_ref[...] + te + de


def _node_sum_tc(attr_rows, nt3, dp3, ttab_p, dtab_p):
    grid = (N_NODES // _NB,)
    return pl.pallas_call(
        _nodesum_body,
        grid=grid,
        in_specs=[
            pl.BlockSpec((_NB, EMB), lambda i: (i, 0)),
            pl.BlockSpec((1, 1, _NB), lambda i: (i, 0, 0)),
            pl.BlockSpec((1, 1, _NB), lambda i: (i, 0, 0)),
            pl.BlockSpec((_TT_PAD, EMB), lambda i: (0, 0)),
            pl.BlockSpec((_DT_PAD, EMB), lambda i: (0, 0)),
        ],
        out_specs=pl.BlockSpec((_NB, EMB), lambda i: (i, 0)),
        out_shape=jax.ShapeDtypeStruct((N_NODES, EMB), jnp.float32),
    )(attr_rows, nt3, dp3, ttab_p, dtab_p)


def _pad1d(x, total):
    return jnp.pad(x, (0, total - x.shape[0]))


def kernel(nodes, depth, edge_type, edge_name, node_type_table,
           node_attr_table, depth_table, edge_df_table, edge_ast_table):
    na2 = _pad1d(nodes[:, 1].astype(jnp.int32), _NP)
    nt3 = nodes[:, 0].astype(jnp.int32).reshape(N_NODES // _NB, 1, _NB)
    dp3 = depth[:, 0].astype(jnp.int32).reshape(N_NODES // _NB, 1, _NB)
    et = edge_type.astype(jnp.int32)
    en = edge_name.astype(jnp.int32)
    ttab_p = jnp.pad(node_type_table, ((0, _TT_PAD - 100), (0, 0)))
    dtab_p = jnp.pad(depth_table, ((0, _DT_PAD - (MAX_DEPTH + 1)), (0, 0)))

    comb = _comb_table(edge_df_table, edge_ast_table)
    edge_out, attr_rows = _edge_sc(et, en, comb, na2, node_attr_table)
    node_out = _node_sum_tc(attr_rows, nt3, dp3, ttab_p, dtab_p)
    return (node_out, edge_out)
